# baseline (speedup 1.0000x reference)
import functools

import jax
import jax.numpy as jnp
from jax.experimental import pallas as pl
from jax.experimental.pallas import tpu as pltpu

_TILE = 10000


def _dot(a, b, dims):
    return jax.lax.dot_general(a, b, (dims, ((), ())),
                               preferred_element_type=jnp.float32)


def _leaky(v):
    return jnp.where(v >= 0, v, 0.01 * v)


def _fused_kernel(x_ref, e_ref, we_ref, be_ref, wsi_ref, bsi_ref,
                  wso_ref, bso_ref, wo_ref, bo_ref,
                  skip_out_ref, to_gnn_ref,
                  xbf_ref, sec1_ref, cs_ref, m_ref, l_ref, acc_ref,
                  mso_ref, mog_ref, vso_ref, vg_ref, *, g, tile):
    i = pl.program_id(0)
    bf16 = jnp.bfloat16

    @pl.when(i < g)
    def _phase_a():
        x = x_ref[...]
        e = e_ref[...]
        xbf_ref[pl.ds(i * tile, tile), :] = x.astype(bf16)
        part = _dot(e, x, ((0,), (0,)))
        ones = jnp.ones((tile, 1), jnp.float32)
        cs_part = _dot(e, ones, ((0,), (0,)))

        @pl.when(i == 0)
        def _():
            sec1_ref[...] = jnp.zeros_like(sec1_ref)
            cs_ref[...] = jnp.zeros_like(cs_ref)

        sec1_ref[...] += part
        cs_ref[...] += cs_part

        @pl.when(i == g - 1)
        def _():
            sec1_ref[...] = sec1_ref[...] / cs_ref[...]

    @pl.when((i >= g) & (i < 2 * g))
    def _phase_b():
        j = i - g
        xb = xbf_ref[pl.ds(j * tile, tile), :]
        s = _dot(sec1_ref[...].astype(bf16), xb, ((1,), (1,)))
        tile_max = jnp.max(s, axis=1, keepdims=True)

        @pl.when(j == 0)
        def _():
            m_ref[...] = jnp.full_like(m_ref, -jnp.inf)
            l_ref[...] = jnp.zeros_like(l_ref)
            acc_ref[...] = jnp.zeros_like(acc_ref)

        m_old = m_ref[...]
        m_new = jnp.maximum(m_old, tile_max)
        corr = jnp.exp(m_old - m_new)
        p = jnp.exp(s - m_new)
        l_ref[...] = l_ref[...] * corr + jnp.sum(p, axis=1, keepdims=True)
        acc_ref[...] = acc_ref[...] * corr + _dot(p.astype(bf16), xb, ((1,), (0,)))
        m_ref[...] = m_new

        @pl.when(j == g - 1)
        def _():
            sec2 = acc_ref[...] / l_ref[...]
            acc_ref[...] = sec2
            t1 = _dot(sec2, we_ref[...], ((1,), (1,)))
            mso_ref[...] = _dot(t1, wso_ref[...], ((1,), (1,)))
            t2 = _dot(t1, wsi_ref[...], ((1,), (1,)))
            mog_ref[...] = _dot(t2, wo_ref[...], ((1,), (1,)))
            vso_ref[...] = _dot(be_ref[...], wso_ref[...], ((1,), (1,))) + bso_ref[...]
            b1 = _dot(be_ref[...], wsi_ref[...], ((1,), (1,))) + bsi_ref[...]
            vg_ref[...] = _dot(b1, wo_ref[...], ((1,), (1,))) + bo_ref[...]

    @pl.when(i >= 2 * g)
    def _phase_c():
        k = i - 2 * g
        xb = xbf_ref[pl.ds(k * tile, tile), :]
        logits = _dot(xb, acc_ref[...].astype(bf16), ((1,), (1,)))
        logits = logits - jnp.max(logits, axis=1, keepdims=True)
        p = jnp.exp(logits)
        inv = (p / jnp.sum(p, axis=1, keepdims=True)).astype(bf16)
        skip_out_ref[...] = _leaky(
            _dot(inv, mso_ref[...].astype(bf16), ((1,), (0,))) + vso_ref[...]
        ).astype(bf16)
        to_gnn_ref[...] = _leaky(
            _dot(xb, wo_ref[...].astype(bf16), ((1,), (1,)))
            + _dot(inv, mog_ref[...].astype(bf16), ((1,), (0,))) + vg_ref[...]
        ).astype(bf16)


@jax.jit
def kernel(x, ent2sec_mat, W_ent, b_ent, W_skip_in, b_skip_in,
           W_skip_out, b_skip_out, W_out, b_out):
    n, d = x.shape
    s = ent2sec_mat.shape[1]
    tile = _TILE if n % _TILE == 0 else n
    g = n // tile

    in_tile = lambda i: (jnp.where(i < g, i, 0), 0)
    out_tile = lambda i: (jnp.where(i >= 2 * g, i - 2 * g, 0), 0)
    whole = lambda i: (0, 0)

    bias2d = lambda b: b.reshape(1, d)
    wspec = pl.BlockSpec((d, d), whole)
    bspec = pl.BlockSpec((1, d), whole)
    sd = lambda: pltpu.VMEM((s, d), jnp.float32)
    s1 = lambda: pltpu.VMEM((s, 1), jnp.float32)

    skip_out, to_gnn = pl.pallas_call(
        functools.partial(_fused_kernel, g=g, tile=tile),
        grid=(3 * g,),
        in_specs=[pl.BlockSpec((tile, d), in_tile),
                  pl.BlockSpec((tile, s), in_tile),
                  wspec, bspec, wspec, bspec, wspec, bspec, wspec, bspec],
        out_specs=[pl.BlockSpec((tile, d), out_tile),
                   pl.BlockSpec((tile, d), out_tile)],
        out_shape=[jax.ShapeDtypeStruct((n, d), jnp.bfloat16),
                   jax.ShapeDtypeStruct((n, d), jnp.bfloat16)],
        scratch_shapes=[pltpu.VMEM((n, d), jnp.bfloat16),
                        sd(), s1(), s1(), s1(), sd(), sd(), sd(),
                        pltpu.VMEM((1, d), jnp.float32),
                        pltpu.VMEM((1, d), jnp.float32)],
    )(x, ent2sec_mat, W_ent, bias2d(b_ent), W_skip_in, bias2d(b_skip_in),
      W_skip_out, bias2d(b_skip_out), W_out, bias2d(b_out))

    return (skip_out.astype(jnp.float32), to_gnn.astype(jnp.float32))
